# baseline (speedup 1.0000x reference)
import functools

import jax
import jax.numpy as jnp
from jax.experimental import pallas as pl

_B = 512
_E = 64


def _body(x_ref, c_ref, r_ref, out_ref, idx_ref):
    x = x_ref[...]
    c = c_ref[...]
    c_norm = jnp.sum(c * c, axis=1)[None, :]
    g = jax.lax.dot_general(
        x, c, (((1,), (1,)), ((), ())),
        preferred_element_type=jnp.float32,
        precision=jax.lax.Precision.HIGHEST)
    scores = c_norm - 2.0 * g
    m = jnp.min(scores, axis=1, keepdims=True)
    lane = jax.lax.broadcasted_iota(jnp.int32, scores.shape, 1)
    idxcol = jnp.min(jnp.where(scores == m, lane, _E), axis=1,
                     keepdims=True)
    onehot = (lane == idxcol).astype(jnp.float32)
    closest = jax.lax.dot_general(
        onehot, c, (((1,), (0,)), ((), ())),
        preferred_element_type=jnp.float32,
        precision=jax.lax.Precision.HIGHEST)
    strength = jax.nn.sigmoid(r_ref[...])
    s = jax.lax.dot_general(
        onehot, strength, (((1,), (1,)), ((), ())),
        preferred_element_type=jnp.float32,
        precision=jax.lax.Precision.HIGHEST)
    out_ref[...] = x + (0.1 * s) * (closest - x)
    lane_f = jax.lax.broadcasted_iota(jnp.int32, (1, _E), 1).astype(jnp.float32)
    idxrow = jax.lax.dot_general(
        lane_f, onehot, (((1,), (1,)), ((), ())),
        preferred_element_type=jnp.float32,
        precision=jax.lax.Precision.HIGHEST)
    idx_ref[...] = idxrow.astype(jnp.int32)[:, None, :]


@jax.jit
def kernel(expert_activations, attractor_centers, attraction_radii):
    batch, e = expert_activations.shape
    grid = batch // _B
    r2d = attraction_radii[None, :]
    out, idx = pl.pallas_call(
        _body,
        grid=(grid,),
        in_specs=[
            pl.BlockSpec((_B, e), lambda i: (i, 0)),
            pl.BlockSpec((e, e), lambda i: (0, 0)),
            pl.BlockSpec((1, e), lambda i: (0, 0)),
        ],
        out_specs=[
            pl.BlockSpec((_B, e), lambda i: (i, 0)),
            pl.BlockSpec((1, 1, _B), lambda i: (i, 0, 0)),
        ],
        out_shape=[
            jax.ShapeDtypeStruct((batch, e), jnp.float32),
            jax.ShapeDtypeStruct((grid, 1, _B), jnp.int32),
        ],
    )(expert_activations, attractor_centers, r2d)
    return (out, idx.reshape(batch))


# speedup vs baseline: 14.0838x; 14.0838x over previous
import functools

import jax
import jax.numpy as jnp
from jax.experimental import pallas as pl

_B = 1024
_E = 64


def _body(x_ref, c_ref, r_ref, out_ref, idx_ref):
    x = x_ref[...]
    c = c_ref[...]
    c_norm = jnp.sum(c * c, axis=1, keepdims=True)
    g = jax.lax.dot_general(
        c, x, (((1,), (1,)), ((), ())),
        preferred_element_type=jnp.float32,
        precision=jax.lax.Precision.HIGHEST)
    scores = c_norm - 2.0 * g
    m = jnp.min(scores, axis=0, keepdims=True)
    subl = jax.lax.broadcasted_iota(jnp.int32, scores.shape, 0)
    idxrow = jnp.min(jnp.where(scores == m, subl, _E), axis=0,
                     keepdims=True)
    onehot_t = (subl == idxrow).astype(jnp.float32)
    w = 0.1 * jax.nn.sigmoid(r_ref[...])
    cs = w * c
    rhs = jnp.concatenate([cs, jnp.broadcast_to(w, (_E, _E))], 1)
    p = jax.lax.dot_general(
        onehot_t, rhs, (((0,), (0,)), ((), ())),
        preferred_element_type=jnp.float32,
        precision=jax.lax.Precision.HIGHEST)
    closest_s = p[:, :_E]
    sfull = p[:, _E:]
    out_ref[...] = x * (1.0 - sfull) + closest_s
    idx_ref[...] = idxrow[:, None, :]


@jax.jit
def kernel(expert_activations, attractor_centers, attraction_radii):
    batch, e = expert_activations.shape
    grid = batch // _B
    r2d = attraction_radii[:, None]
    out, idx = pl.pallas_call(
        _body,
        grid=(grid,),
        in_specs=[
            pl.BlockSpec((_B, e), lambda i: (i, 0)),
            pl.BlockSpec((e, e), lambda i: (0, 0)),
            pl.BlockSpec((e, 1), lambda i: (0, 0)),
        ],
        out_specs=[
            pl.BlockSpec((_B, e), lambda i: (i, 0)),
            pl.BlockSpec((1, 1, _B), lambda i: (i, 0, 0)),
        ],
        out_shape=[
            jax.ShapeDtypeStruct((batch, e), jnp.float32),
            jax.ShapeDtypeStruct((grid, 1, _B), jnp.int32),
        ],
    )(expert_activations, attractor_centers, r2d)
    return (out, idx.reshape(batch))


# speedup vs baseline: 15.7401x; 1.1176x over previous
import functools

import jax
import jax.numpy as jnp
from jax.experimental import pallas as pl

_B = 2048
_E = 64


def _body(x_ref, c_ref, r_ref, out_ref, idx_ref):
    x = x_ref[...]
    c = c_ref[...]
    c_norm = jnp.sum(c * c, axis=1, keepdims=True)
    g = jax.lax.dot_general(
        c, x, (((1,), (1,)), ((), ())),
        preferred_element_type=jnp.float32,
        precision=jax.lax.Precision.HIGHEST)
    scores = c_norm - 2.0 * g
    m = jnp.min(scores, axis=0, keepdims=True)
    subl = jax.lax.broadcasted_iota(jnp.int32, scores.shape, 0)
    idxrow = jnp.min(jnp.where(scores == m, subl, _E), axis=0,
                     keepdims=True)
    onehot_t = (subl == idxrow).astype(jnp.float32)
    w = 0.1 * jax.nn.sigmoid(r_ref[...])
    cs = w * c
    rhs = jnp.concatenate([cs, jnp.broadcast_to(w, (_E, _E))], 1)
    p = jax.lax.dot_general(
        onehot_t, rhs, (((0,), (0,)), ((), ())),
        preferred_element_type=jnp.float32,
        precision=jax.lax.Precision.HIGHEST)
    closest_s = p[:, :_E]
    sfull = p[:, _E:]
    out_ref[...] = x * (1.0 - sfull) + closest_s
    idx_ref[...] = idxrow[:, None, :]


@jax.jit
def kernel(expert_activations, attractor_centers, attraction_radii):
    batch, e = expert_activations.shape
    grid = batch // _B
    r2d = attraction_radii[:, None]
    out, idx = pl.pallas_call(
        _body,
        grid=(grid,),
        in_specs=[
            pl.BlockSpec((_B, e), lambda i: (i, 0)),
            pl.BlockSpec((e, e), lambda i: (0, 0)),
            pl.BlockSpec((e, 1), lambda i: (0, 0)),
        ],
        out_specs=[
            pl.BlockSpec((_B, e), lambda i: (i, 0)),
            pl.BlockSpec((1, 1, _B), lambda i: (i, 0, 0)),
        ],
        out_shape=[
            jax.ShapeDtypeStruct((batch, e), jnp.float32),
            jax.ShapeDtypeStruct((grid, 1, _B), jnp.int32),
        ],
    )(expert_activations, attractor_centers, r2d)
    return (out, idx.reshape(batch))


# speedup vs baseline: 16.0016x; 1.0166x over previous
import functools

import jax
import jax.numpy as jnp
from jax.experimental import pallas as pl

_B = 4096
_E = 64


def _body(x_ref, c_ref, r_ref, out_ref, idx_ref):
    x = x_ref[...]
    c = c_ref[...]
    c_norm = jnp.sum(c * c, axis=1, keepdims=True)
    g = jax.lax.dot_general(
        c, x, (((1,), (1,)), ((), ())),
        preferred_element_type=jnp.float32,
        precision=jax.lax.Precision.HIGHEST)
    scores = c_norm - 2.0 * g
    m = jnp.min(scores, axis=0, keepdims=True)
    subl = jax.lax.broadcasted_iota(jnp.int32, scores.shape, 0)
    idxrow = jnp.min(jnp.where(scores == m, subl, _E), axis=0,
                     keepdims=True)
    onehot_t = (subl == idxrow).astype(jnp.float32)
    w = 0.1 * jax.nn.sigmoid(r_ref[...])
    cs = w * c
    rhs = jnp.concatenate([cs, jnp.broadcast_to(w, (_E, _E))], 1)
    p = jax.lax.dot_general(
        onehot_t, rhs, (((0,), (0,)), ((), ())),
        preferred_element_type=jnp.float32,
        precision=jax.lax.Precision.HIGHEST)
    closest_s = p[:, :_E]
    sfull = p[:, _E:]
    out_ref[...] = x * (1.0 - sfull) + closest_s
    idx_ref[...] = idxrow[:, None, :]


@jax.jit
def kernel(expert_activations, attractor_centers, attraction_radii):
    batch, e = expert_activations.shape
    grid = batch // _B
    r2d = attraction_radii[:, None]
    out, idx = pl.pallas_call(
        _body,
        grid=(grid,),
        in_specs=[
            pl.BlockSpec((_B, e), lambda i: (i, 0)),
            pl.BlockSpec((e, e), lambda i: (0, 0)),
            pl.BlockSpec((e, 1), lambda i: (0, 0)),
        ],
        out_specs=[
            pl.BlockSpec((_B, e), lambda i: (i, 0)),
            pl.BlockSpec((1, 1, _B), lambda i: (i, 0, 0)),
        ],
        out_shape=[
            jax.ShapeDtypeStruct((batch, e), jnp.float32),
            jax.ShapeDtypeStruct((grid, 1, _B), jnp.int32),
        ],
    )(expert_activations, attractor_centers, r2d)
    return (out, idx.reshape(batch))


# speedup vs baseline: 18.5002x; 1.1561x over previous
import jax
import jax.numpy as jnp
from jax.experimental import pallas as pl

_B = 4096
_E = 64


def _body(x_ref, c_ref, r_ref, out_ref, idx_ref):
    x = x_ref[...]
    c = c_ref[...]
    c_norm = jnp.sum(c * c, axis=1, keepdims=True)
    g = jax.lax.dot_general(
        c, x, (((1,), (1,)), ((), ())),
        preferred_element_type=jnp.float32,
        precision=jax.lax.Precision.HIGHEST)
    scores = c_norm - 2.0 * g
    m = jnp.min(scores, axis=0, keepdims=True)
    subl = jax.lax.broadcasted_iota(jnp.int32, scores.shape, 0)
    idxrow = jnp.min(jnp.where(scores == m, subl, _E), axis=0,
                     keepdims=True)
    onehot_t = (subl == idxrow).astype(jnp.float32)
    w = 0.1 * jax.nn.sigmoid(r_ref[...])
    cs = w * c
    rhs = jnp.concatenate([cs, jnp.broadcast_to(w, (_E, _E))], 1)
    p = jax.lax.dot_general(
        onehot_t, rhs, (((0,), (0,)), ((), ())),
        preferred_element_type=jnp.float32,
        precision=jax.lax.Precision.DEFAULT)
    closest_s = p[:, :_E]
    sfull = p[:, _E:]
    out_ref[...] = x * (1.0 - sfull) + closest_s
    idx_ref[...] = idxrow[:, None, :]


@jax.jit
def kernel(expert_activations, attractor_centers, attraction_radii):
    batch, e = expert_activations.shape
    grid = batch // _B
    r2d = attraction_radii[:, None]
    out, idx = pl.pallas_call(
        _body,
        grid=(grid,),
        in_specs=[
            pl.BlockSpec((_B, e), lambda i: (i, 0)),
            pl.BlockSpec((e, e), lambda i: (0, 0)),
            pl.BlockSpec((e, 1), lambda i: (0, 0)),
        ],
        out_specs=[
            pl.BlockSpec((_B, e), lambda i: (i, 0)),
            pl.BlockSpec((1, 1, _B), lambda i: (i, 0, 0)),
        ],
        out_shape=[
            jax.ShapeDtypeStruct((batch, e), jnp.float32),
            jax.ShapeDtypeStruct((grid, 1, _B), jnp.int32),
        ],
    )(expert_activations, attractor_centers, r2d)
    return (out, idx.reshape(batch))
